# baseline (speedup 1.0000x reference)
import functools

import jax
import jax.numpy as jnp
from jax import lax
from jax.experimental import pallas as pl
from jax.experimental.pallas import tpu as pltpu
from jax.experimental.pallas import tpu_sc as plsc

BATCH = 16384
N_ITEMS = 1000000

_NS = 16
_LANES = 16

_B_PER_W = BATCH // _NS
_CHUNK = 1024
_N_CHUNKS = _B_PER_W // _CHUNK


@functools.partial(
    pl.kernel,
    mesh=plsc.VectorSubcoreMesh(
        core_axis_name="c", subcore_axis_name="s", num_cores=1
    ),
    out_type=jax.ShapeDtypeStruct((BATCH,), jnp.float32),
    scratch_types=[
        pltpu.VMEM((_B_PER_W,), jnp.int32),
        pltpu.VMEM((_B_PER_W,), jnp.float32),
        pltpu.VMEM((_B_PER_W,), jnp.float32),
        pltpu.SemaphoreType.DMA((_N_CHUNKS,)),
        pltpu.SemaphoreType.DMA((_N_CHUNKS,)),
        pltpu.SemaphoreType.DMA,
    ],
)
def _sc_lookup_sigmoid(idx_hbm, table_hbm, out_hbm, idx_v, rows_v, res_v,
                       isems, gsems, osem):
    wid = lax.axis_index("s")
    base = wid * _B_PER_W

    idx_copies = []
    for j in range(_N_CHUNKS):
        sl = pl.ds(j * _CHUNK, _CHUNK)
        idx_copies.append(
            pltpu.async_copy(
                idx_hbm.at[pl.ds(base + j * _CHUNK, _CHUNK)],
                idx_v.at[sl],
                isems.at[j],
            )
        )

    gathers = []
    for j in range(_N_CHUNKS):
        sl = pl.ds(j * _CHUNK, _CHUNK)
        idx_copies[j].wait()
        gathers.append(
            pltpu.async_copy(
                table_hbm.at[idx_v.at[sl]], rows_v.at[sl], gsems.at[j]
            )
        )

    outs = []
    for j in range(_N_CHUNKS):
        gathers[j].wait()
        sls = [
            pl.ds(j * _CHUNK + i * _LANES, _LANES)
            for i in range(_CHUNK // _LANES)
        ]
        xs = [rows_v[sl] for sl in sls]
        es = [jnp.exp(-x) for x in xs]
        rs = [1.0 / (1.0 + e) for e in es]
        for sl, r in zip(sls, rs):
            res_v[sl] = r
        outs.append(
            pltpu.async_copy(
                res_v.at[pl.ds(j * _CHUNK, _CHUNK)],
                out_hbm.at[pl.ds(base + j * _CHUNK, _CHUNK)],
                osem,
            )
        )
    for c in outs:
        c.wait()


def kernel(user_ids, item_ids, item_emb_weight):
    del user_ids
    idx = item_ids.astype(jnp.int32)
    table = item_emb_weight.reshape((N_ITEMS,))
    return _sc_lookup_sigmoid(idx, table)


# speedup vs baseline: 1.0061x; 1.0061x over previous
import functools

import jax
import jax.numpy as jnp
from jax import lax
from jax.experimental import pallas as pl
from jax.experimental.pallas import tpu as pltpu
from jax.experimental.pallas import tpu_sc as plsc

BATCH = 16384
N_ITEMS = 1000000

_NS = 16
_LANES = 16

_B_PER_W = BATCH // _NS
_CHUNKS = (768, 256)
_OFFS = tuple(sum(_CHUNKS[:j]) for j in range(len(_CHUNKS)))
assert sum(_CHUNKS) == _B_PER_W
_N_CHUNKS = len(_CHUNKS)


@functools.partial(
    pl.kernel,
    mesh=plsc.VectorSubcoreMesh(
        core_axis_name="c", subcore_axis_name="s", num_cores=1
    ),
    out_type=jax.ShapeDtypeStruct((BATCH,), jnp.float32),
    scratch_types=[
        pltpu.VMEM((_B_PER_W,), jnp.int32),
        pltpu.VMEM((_B_PER_W,), jnp.float32),
        pltpu.VMEM((_B_PER_W,), jnp.float32),
        pltpu.SemaphoreType.DMA((_N_CHUNKS,)),
        pltpu.SemaphoreType.DMA((_N_CHUNKS,)),
        pltpu.SemaphoreType.DMA,
    ],
)
def _sc_lookup_sigmoid(idx_hbm, table_hbm, out_hbm, idx_v, rows_v, res_v,
                       isems, gsems, osem):
    wid = lax.axis_index("s")
    base = wid * _B_PER_W

    idx_copies = []
    for j in range(_N_CHUNKS):
        sl = pl.ds(_OFFS[j], _CHUNKS[j])
        idx_copies.append(
            pltpu.async_copy(
                idx_hbm.at[pl.ds(base + _OFFS[j], _CHUNKS[j])],
                idx_v.at[sl],
                isems.at[j],
            )
        )

    gathers = []
    for j in range(_N_CHUNKS):
        sl = pl.ds(_OFFS[j], _CHUNKS[j])
        idx_copies[j].wait()
        gathers.append(
            pltpu.async_copy(
                table_hbm.at[idx_v.at[sl]], rows_v.at[sl], gsems.at[j]
            )
        )

    outs = []
    for j in range(_N_CHUNKS):
        gathers[j].wait()
        sls = [
            pl.ds(_OFFS[j] + i * _LANES, _LANES)
            for i in range(_CHUNKS[j] // _LANES)
        ]
        xs = [rows_v[sl] for sl in sls]
        es = [jnp.exp(-x) for x in xs]
        rs = [1.0 / (1.0 + e) for e in es]
        for sl, r in zip(sls, rs):
            res_v[sl] = r
        outs.append(
            pltpu.async_copy(
                res_v.at[pl.ds(_OFFS[j], _CHUNKS[j])],
                out_hbm.at[pl.ds(base + _OFFS[j], _CHUNKS[j])],
                osem,
            )
        )
    for c in outs:
        c.wait()


def kernel(user_ids, item_ids, item_emb_weight):
    del user_ids
    idx = item_ids.astype(jnp.int32)
    table = item_emb_weight.reshape((N_ITEMS,))
    return _sc_lookup_sigmoid(idx, table)


# speedup vs baseline: 1.0079x; 1.0018x over previous
import functools

import jax
import jax.numpy as jnp
from jax import lax
from jax.experimental import pallas as pl
from jax.experimental.pallas import tpu as pltpu
from jax.experimental.pallas import tpu_sc as plsc

BATCH = 16384
N_ITEMS = 1000000

_NS = 16
_LANES = 16

_B_PER_W = BATCH // _NS
_CHUNKS = (512, 512)
_OFFS = tuple(sum(_CHUNKS[:j]) for j in range(len(_CHUNKS)))
assert sum(_CHUNKS) == _B_PER_W
_N_CHUNKS = len(_CHUNKS)


@functools.partial(
    pl.kernel,
    mesh=plsc.VectorSubcoreMesh(
        core_axis_name="c", subcore_axis_name="s", num_cores=1
    ),
    out_type=jax.ShapeDtypeStruct((BATCH,), jnp.float32),
    scratch_types=[
        pltpu.VMEM((_B_PER_W,), jnp.int32),
        pltpu.VMEM((_B_PER_W,), jnp.float32),
        pltpu.VMEM((_B_PER_W,), jnp.float32),
        pltpu.SemaphoreType.DMA((_N_CHUNKS,)),
        pltpu.SemaphoreType.DMA((_N_CHUNKS,)),
        pltpu.SemaphoreType.DMA,
    ],
)
def _sc_lookup_sigmoid(idx_hbm, table_hbm, out_hbm, idx_v, rows_v, res_v,
                       isems, gsems, osem):
    wid = lax.axis_index("s")
    base = wid * _B_PER_W

    idx_copies = []
    for j in range(_N_CHUNKS):
        sl = pl.ds(_OFFS[j], _CHUNKS[j])
        idx_copies.append(
            pltpu.async_copy(
                idx_hbm.at[pl.ds(base + _OFFS[j], _CHUNKS[j])],
                idx_v.at[sl],
                isems.at[j],
            )
        )

    gathers = []
    for j in range(_N_CHUNKS):
        sl = pl.ds(_OFFS[j], _CHUNKS[j])
        idx_copies[j].wait()
        gathers.append(
            pltpu.async_copy(
                table_hbm.at[idx_v.at[sl]], rows_v.at[sl], gsems.at[j]
            )
        )

    outs = []
    for j in range(_N_CHUNKS):
        gathers[j].wait()
        sls = [
            pl.ds(_OFFS[j] + i * _LANES, _LANES)
            for i in range(_CHUNKS[j] // _LANES)
        ]
        xs = [rows_v[sl] for sl in sls]
        es = [jnp.exp(-x) for x in xs]
        rs = [1.0 / (1.0 + e) for e in es]
        for sl, r in zip(sls, rs):
            res_v[sl] = r
        outs.append(
            pltpu.async_copy(
                res_v.at[pl.ds(_OFFS[j], _CHUNKS[j])],
                out_hbm.at[pl.ds(base + _OFFS[j], _CHUNKS[j])],
                osem,
            )
        )
    for c in outs:
        c.wait()


def kernel(user_ids, item_ids, item_emb_weight):
    del user_ids
    idx = item_ids.astype(jnp.int32)
    table = item_emb_weight.reshape((N_ITEMS,))
    return _sc_lookup_sigmoid(idx, table)
